# baseline (speedup 1.0000x reference)
import jax
import jax.numpy as jnp
from jax.experimental import pallas as pl
from jax.experimental.pallas import tpu as pltpu

N_ROWS = 4096
N_COLS = 200
BLOCK_C = 128
N_CHUNKS = N_COLS // 8


def _adrmse_kernel(x_ref, tril_ref, out_ref, kscratch_ref):
    xt = x_ref[...].T

    bits = jax.lax.bitcast_convert_type(xt, jnp.int32)
    kt = bits ^ ((bits >> 31) & jnp.int32(0x7FFFFFFF))
    kscratch_ref[...] = kt
    kt_cs = [kt[8 * c:8 * c + 8, :] for c in range(N_CHUNKS)]

    sub_iota = jax.lax.broadcasted_iota(jnp.int32, (8, BLOCK_C), 0)
    accs = [jnp.zeros((8, BLOCK_C), jnp.int32) for _ in range(N_CHUNKS)]
    for cj in range(N_CHUNKS):
        def body(jj, carry, cj=cj):
            bc = jnp.broadcast_to(
                kscratch_ref[pl.ds(8 * cj + jj, 1), :], (8, BLOCK_C))
            bcp = bc + 1
            w_diag = jnp.where(sub_iota > jj, bcp, bc)
            out = []
            for c in range(N_CHUNKS):
                w = bc if c < cj else (bcp if c > cj else w_diag)
                out.append(carry[c] + (w > kt_cs[c]).astype(jnp.int32))
            return out
        accs = jax.lax.fori_loop(0, 8, body, accs, unroll=8)
    rank = 1.0 + jnp.concatenate(accs, axis=0).astype(jnp.float32)

    m = jnp.max(xt, axis=0, keepdims=True)
    e = jnp.exp(xt - m)
    p = e / jnp.sum(e, axis=0, keepdims=True)
    ar = jax.lax.dot(tril_ref[...], p)

    d = (rank - ar) ** 2 / jnp.log2(rank + 1.0)
    partial = jnp.sum(d)

    @pl.when(pl.program_id(0) == 0)
    def _():
        out_ref[0, 0] = 0.0

    out_ref[0, 0] += partial


@jax.jit
def kernel(scores):
    ii = jax.lax.broadcasted_iota(jnp.int32, (N_COLS, N_COLS), 0)
    jj = jax.lax.broadcasted_iota(jnp.int32, (N_COLS, N_COLS), 1)
    tril = (ii >= jj).astype(jnp.float32)
    total = pl.pallas_call(
        _adrmse_kernel,
        grid=(N_ROWS // BLOCK_C,),
        in_specs=[
            pl.BlockSpec((BLOCK_C, N_COLS), lambda i: (i, 0)),
            pl.BlockSpec((N_COLS, N_COLS), lambda i: (0, 0)),
        ],
        out_specs=pl.BlockSpec(memory_space=pltpu.SMEM),
        out_shape=jax.ShapeDtypeStruct((1, 1), jnp.float32),
        scratch_shapes=[pltpu.VMEM((N_COLS, BLOCK_C), jnp.int32)],
    )(scores, tril)
    return total[0, 0] / (N_ROWS * N_COLS)


# speedup vs baseline: 1.1061x; 1.1061x over previous
import jax
import jax.numpy as jnp
from jax.experimental import pallas as pl
from jax.experimental.pallas import tpu as pltpu

N_ROWS = 4096
N_COLS = 200
BLOCK_C = 128
N_CHUNKS = N_COLS // 8


def _adrmse_kernel(xt_ref, out_ref, kscratch_ref, tril_ref):
    @pl.when(pl.program_id(0) == 0)
    def _():
        ii = jax.lax.broadcasted_iota(jnp.int32, (N_COLS, N_COLS), 0)
        jj = jax.lax.broadcasted_iota(jnp.int32, (N_COLS, N_COLS), 1)
        tril_ref[...] = (ii >= jj).astype(jnp.float32)
        out_ref[0, 0] = 0.0

    xt = xt_ref[...]

    bits = jax.lax.bitcast_convert_type(xt, jnp.int32)
    kt = bits ^ ((bits >> 31) & jnp.int32(0x7FFFFFFF))
    kscratch_ref[...] = kt
    kt_cs = [kt[8 * c:8 * c + 8, :] for c in range(N_CHUNKS)]

    sub_iota = jax.lax.broadcasted_iota(jnp.int32, (8, BLOCK_C), 0)
    accs = [jnp.zeros((8, BLOCK_C), jnp.int32) for _ in range(N_CHUNKS)]
    for cj in range(N_CHUNKS):
        def body(jj, carry, cj=cj):
            bc = jnp.broadcast_to(
                kscratch_ref[pl.ds(8 * cj + jj, 1), :], (8, BLOCK_C))
            bcp = bc + 1
            w_diag = jnp.where(sub_iota > jj, bcp, bc)
            out = []
            for c in range(N_CHUNKS):
                w = bc if c < cj else (bcp if c > cj else w_diag)
                out.append(carry[c] + (w > kt_cs[c]).astype(jnp.int32))
            return out
        accs = jax.lax.fori_loop(0, 8, body, accs, unroll=8)
    rank = 1.0 + jnp.concatenate(accs, axis=0).astype(jnp.float32)

    m = jnp.max(xt, axis=0, keepdims=True)
    e = jnp.exp(xt - m)
    p = e / jnp.sum(e, axis=0, keepdims=True)
    ar = jax.lax.dot(tril_ref[...], p)

    d = (rank - ar) ** 2 / jnp.log2(rank + 1.0)
    out_ref[0, 0] += jnp.sum(d)


@jax.jit
def kernel(scores):
    xt = scores.T
    total = pl.pallas_call(
        _adrmse_kernel,
        grid=(N_ROWS // BLOCK_C,),
        in_specs=[pl.BlockSpec((N_COLS, BLOCK_C), lambda i: (0, i))],
        out_specs=pl.BlockSpec(memory_space=pltpu.SMEM),
        out_shape=jax.ShapeDtypeStruct((1, 1), jnp.float32),
        scratch_shapes=[
            pltpu.VMEM((N_COLS, BLOCK_C), jnp.int32),
            pltpu.VMEM((N_COLS, N_COLS), jnp.float32),
        ],
    )(xt)
    return total[0, 0] / (N_ROWS * N_COLS)
